# Initial kernel scaffold; baseline (speedup 1.0000x reference)
import jax
import jax.numpy as jnp
from jax.experimental import pallas as pl


def kernel(x, edge_index, edge_weight, batch, W1, b1, W2, b2, W3, b3, bn_gamma, bn_beta, lin_W, lin_b):
    raise NotImplementedError("write your pallas kernel here")


# speedup vs baseline: 4.6404x; 4.6404x over previous
import functools

import jax
import jax.numpy as jnp
from jax import lax
from jax.experimental import pallas as pl
from jax.experimental.pallas import tpu as pltpu
from jax.experimental.pallas import tpu_sc as plsc

N = 10000
E = 320000
F = 128
G = 64

NC = 2
NS = 16
NW = NC * NS
LANES = 16

EB = 128
NUM_CHUNKS = E // EB
STRIPE = 624
TAIL = N - NS * STRIPE
ZROWS = 208
FV = F // LANES


def _sc_aggregate_body(src_hbm, dst_hbm, w_hbm, h_hbm, out_hbm,
                       acc_sh, src_v, dst_v, w_v, rows_v, zbuf, sem):
    cid = lax.axis_index("c")
    sid = lax.axis_index("s")
    wid = cid * NS + sid

    def zrow(r, _):
        for k in range(FV):
            zbuf[r, pl.ds(k * LANES, LANES)] = jnp.zeros((LANES,), jnp.float32)
        return 0
    lax.fori_loop(0, ZROWS, zrow, 0)
    for j in range(STRIPE // ZROWS):
        pltpu.sync_copy(zbuf, acc_sh.at[pl.ds(sid * STRIPE + j * ZROWS, ZROWS)])
    @pl.when(sid == NS - 1)
    def _():
        pltpu.sync_copy(zbuf.at[pl.ds(0, TAIL)], acc_sh.at[pl.ds(NS * STRIPE, TAIL)])
    plsc.subcore_barrier()

    n_chunks = (NUM_CHUNKS - wid + NW - 1) // NW

    def chunk(i, _):
        base = (wid + i * NW) * EB
        pltpu.sync_copy(src_hbm.at[pl.ds(base, EB)], src_v)
        pltpu.sync_copy(dst_hbm.at[pl.ds(base, EB)], dst_v)
        pltpu.sync_copy(w_hbm.at[pl.ds(base, EB)], w_v)
        pltpu.async_copy(h_hbm.at[src_v], rows_v, sem).wait()

        def group(gj, _):
            w16 = w_v[pl.ds(gj * LANES, LANES)]
            for i16 in range(LANES):
                wj = w16[i16]
                j = gj * LANES + i16
                for k in range(FV):
                    s = pl.ds(k * LANES, LANES)
                    rows_v[j, s] = rows_v[j, s] * wj
            return 0
        lax.fori_loop(0, EB // LANES, group, 0)
        pltpu.sync_copy(rows_v, acc_sh.at[dst_v], add=True)
        return 0

    lax.fori_loop(0, n_chunks, chunk, 0)
    plsc.subcore_barrier()

    pltpu.sync_copy(acc_sh.at[pl.ds(sid * STRIPE, STRIPE)],
                    out_hbm.at[cid, pl.ds(sid * STRIPE, STRIPE)])
    @pl.when(sid == NS - 1)
    def _():
        pltpu.sync_copy(acc_sh.at[pl.ds(NS * STRIPE, TAIL)],
                        out_hbm.at[cid, pl.ds(NS * STRIPE, TAIL)])


def _sc_aggregate(src, dst, w, h):
    mesh = plsc.VectorSubcoreMesh(core_axis_name="c", subcore_axis_name="s",
                                  num_cores=NC, num_subcores=NS)
    f = pl.kernel(
        _sc_aggregate_body,
        out_type=jax.ShapeDtypeStruct((NC, N, F), jnp.float32),
        mesh=mesh,
        scratch_types=[
            pltpu.VMEM_SHARED((N, F), jnp.float32),
            pltpu.VMEM((EB,), jnp.int32),
            pltpu.VMEM((EB,), jnp.int32),
            pltpu.VMEM((EB,), jnp.float32),
            pltpu.VMEM((EB, F), jnp.float32),
            pltpu.VMEM((ZROWS, F), jnp.float32),
            pltpu.SemaphoreType.DMA,
        ],
    )
    return f(src, dst, w, h)


GPW = G // NW
SEG_CHUNK = 64
SEG_BUF = SEG_CHUNK + 8


def _sc_segmax_body(p_hbm, batch_hbm, b3_hbm, out_hbm,
                    batch_v, buf0, buf1, b3_v, obuf):
    cid = lax.axis_index("c")
    sid = lax.axis_index("s")
    wid = cid * NS + sid
    g0 = wid * GPW

    pltpu.sync_copy(batch_hbm, batch_v)
    pltpu.sync_copy(b3_hbm, b3_v)

    def cnt(i, carry):
        v = batch_v[pl.ds(i * LANES, LANES)]
        out = []
        for t in range(GPW + 1):
            m = jnp.minimum(jnp.maximum((g0 + t) - v, 0), 1)
            out.append(carry[t] + m)
        return tuple(out)
    zeros = tuple(jnp.zeros((LANES,), jnp.int32) for _ in range(GPW + 1))
    counts = lax.fori_loop(0, N // LANES, cnt, zeros)
    bounds = [sum(c[i] for i in range(LANES)) for c in counts]

    for t in range(GPW):
        s_row = bounds[t]
        e_row = bounds[t + 1]
        acc0 = tuple(jnp.full((LANES,), -jnp.inf, jnp.float32) for _ in range(FV))
        nch = (e_row - s_row + SEG_CHUNK - 1) // SEG_CHUNK

        def chunk(i, acc, s_row=s_row, e_row=e_row):
            base = s_row + i * SEG_CHUNK
            db = jnp.minimum((base // 8) * 8, N - SEG_BUF)
            pltpu.sync_copy(p_hbm.at[0, pl.ds(db, SEG_BUF)], buf0)
            pltpu.sync_copy(p_hbm.at[1, pl.ds(db, SEG_BUF)], buf1)
            off = base - db
            cnt_rows = jnp.minimum(e_row - base, SEG_CHUNK)

            def row(r, acc):
                rr = off + r
                new = []
                for k in range(FV):
                    sl = pl.ds(k * LANES, LANES)
                    new.append(jnp.maximum(acc[k], buf0[rr, sl] + buf1[rr, sl]))
                return tuple(new)
            return lax.fori_loop(0, cnt_rows, row, acc)

        acc = lax.fori_loop(0, nch, chunk, acc0)
        for k in range(FV):
            obuf[pl.ds(t * F + k * LANES, LANES)] = acc[k] + b3_v[pl.ds(k * LANES, LANES)]
    pltpu.sync_copy(obuf, out_hbm.at[pl.ds(wid * GPW * F, GPW * F)])


def _sc_segmax(p, batch, b3):
    mesh = plsc.VectorSubcoreMesh(core_axis_name="c", subcore_axis_name="s",
                                  num_cores=NC, num_subcores=NS)
    f = pl.kernel(
        _sc_segmax_body,
        out_type=jax.ShapeDtypeStruct((G * F,), jnp.float32),
        mesh=mesh,
        scratch_types=[
            pltpu.VMEM((N,), jnp.int32),
            pltpu.VMEM((SEG_BUF, F), jnp.float32),
            pltpu.VMEM((SEG_BUF, F), jnp.float32),
            pltpu.VMEM((F,), jnp.float32),
            pltpu.VMEM((GPW * F,), jnp.float32),
        ],
    )
    return f(p, batch, b3).reshape(G, F)


MM_ROWS = 2000


def _mm_first_body(x_ref, w_ref, o_ref):
    o_ref[...] = jnp.dot(x_ref[...], w_ref[...], preferred_element_type=jnp.float32)


def _mm_first(x, w):
    grid = (N // MM_ROWS,)
    return pl.pallas_call(
        _mm_first_body,
        grid=grid,
        in_specs=[
            pl.BlockSpec((MM_ROWS, F), lambda i: (i, 0)),
            pl.BlockSpec((F, F), lambda i: (0, 0)),
        ],
        out_specs=pl.BlockSpec((MM_ROWS, F), lambda i: (i, 0)),
        out_shape=jax.ShapeDtypeStruct((N, F), jnp.float32),
    )(x, w)


def _mm_fused_body(p_ref, b_ref, w_ref, o_ref):
    h = jax.nn.relu(p_ref[0] + p_ref[1] + b_ref[...])
    o_ref[...] = jnp.dot(h, w_ref[...], preferred_element_type=jnp.float32)


def _mm_fused(p, b, w):
    grid = (N // MM_ROWS,)
    return pl.pallas_call(
        _mm_fused_body,
        grid=grid,
        in_specs=[
            pl.BlockSpec((NC, MM_ROWS, F), lambda i: (0, i, 0)),
            pl.BlockSpec((1, F), lambda i: (0, 0)),
            pl.BlockSpec((F, F), lambda i: (0, 0)),
        ],
        out_specs=pl.BlockSpec((MM_ROWS, F), lambda i: (i, 0)),
        out_shape=jax.ShapeDtypeStruct((N, F), jnp.float32),
    )(p, b.reshape(1, F), w)


def _head_body(g_ref, gam_ref, bet_ref, lw_ref, lb_ref, o_ref):
    g = g_ref[...]
    mean = jnp.mean(g, axis=0, keepdims=True)
    var = jnp.mean((g - mean) ** 2, axis=0, keepdims=True)
    gn = (g - mean) * lax.rsqrt(var + 1e-5) * gam_ref[...] + bet_ref[...]
    o_ref[...] = jnp.sum(gn * lw_ref[...], axis=1, keepdims=True) + lb_ref[...]


def _head(g, gamma, beta, lin_W, lin_b):
    return pl.pallas_call(
        _head_body,
        out_shape=jax.ShapeDtypeStruct((G, 1), jnp.float32),
    )(g, gamma.reshape(1, F), beta.reshape(1, F), lin_W.reshape(1, F),
      lin_b.reshape(1, 1))


def kernel(x, edge_index, edge_weight, batch, W1, b1, W2, b2, W3, b3,
           bn_gamma, bn_beta, lin_W, lin_b):
    src = edge_index[0]
    dst = edge_index[1]
    h = _mm_first(x, W1)
    p = _sc_aggregate(src, dst, edge_weight, h)
    h = _mm_fused(p, b1, W2)
    p = _sc_aggregate(src, dst, edge_weight, h)
    h = _mm_fused(p, b2, W3)
    p = _sc_aggregate(src, dst, edge_weight, h)
    g = _sc_segmax(p, batch, b3)
    return _head(g, bn_gamma, bn_beta, lin_W, lin_b)


# speedup vs baseline: 5.9341x; 1.2788x over previous
import functools

import jax
import jax.numpy as jnp
from jax import lax
from jax.experimental import pallas as pl
from jax.experimental.pallas import tpu as pltpu
from jax.experimental.pallas import tpu_sc as plsc

N = 10000
E = 320000
F = 128
G = 64

NC = 2
NS = 16
NW = NC * NS
LANES = 16

EB = 128
NUM_CHUNKS = E // EB
STRIPE = 624
TAIL = N - NS * STRIPE
ZROWS = 208
FV = F // LANES


NCH = (NUM_CHUNKS + NW - 1) // NW
EPAD = NW * NCH * EB
DEPTH = 3


def _sc_aggregate_body(esd_hbm, w_hbm, h_hbm, out_hbm, acc_sh, *rest):
    esds = rest[0:DEPTH]
    rowss = rest[DEPTH:2 * DEPTH]
    dstbs = rest[2 * DEPTH:3 * DEPTH]
    wbufs = rest[3 * DEPTH:4 * DEPTH]
    isems = rest[4 * DEPTH:5 * DEPTH]
    gsems = rest[5 * DEPTH:6 * DEPTH]
    ssems = rest[6 * DEPTH:7 * DEPTH]

    cid = lax.axis_index("c")
    sid = lax.axis_index("s")
    wid = cid * NS + sid

    def idx_src(c):
        return esd_hbm.at[:, pl.ds((wid + c * NW) * EB, EB)]

    def w_src(c):
        return w_hbm.at[pl.ds((wid + c * NW) * EB, EB)]

    def idx_start(c, b):
        pltpu.async_copy(idx_src(c), esds[b], isems[b])
        pltpu.async_copy(w_src(c), wbufs[b], isems[b])

    def idx_wait(c, b):
        pltpu.make_async_copy(idx_src(c), esds[b], isems[b]).wait()
        pltpu.make_async_copy(w_src(c), wbufs[b], isems[b]).wait()

    idx_start(0, 0)

    def zrow(r, _):
        for k in range(FV):
            rowss[0][r, pl.ds(k * LANES, LANES)] = jnp.zeros((LANES,), jnp.float32)
        return 0
    lax.fori_loop(0, EB, zrow, 0)
    for j in range(STRIPE // EB):
        pltpu.sync_copy(rowss[0], acc_sh.at[pl.ds(sid * STRIPE + j * EB, EB)])
    rem = STRIPE - (STRIPE // EB) * EB
    pltpu.sync_copy(rowss[0].at[pl.ds(0, rem)],
                    acc_sh.at[pl.ds(sid * STRIPE + (STRIPE // EB) * EB, rem)])
    @pl.when(sid == NS - 1)
    def _():
        pltpu.sync_copy(rowss[0].at[pl.ds(0, TAIL)],
                        acc_sh.at[pl.ds(NS * STRIPE, TAIL)])
    plsc.subcore_barrier()

    def slot(i, b):
        c = i * DEPTH + b

        @pl.when(jnp.logical_and(c < NCH, c >= DEPTH))
        def _():
            pltpu.make_async_copy(rowss[b], acc_sh.at[dstbs[b]], ssems[b]).wait()

        @pl.when(c < NCH)
        def _():
            idx_wait(c, b)
            pltpu.async_copy(h_hbm.at[esds[b].at[0]], rowss[b], gsems[b])

        c2 = c - 2
        b2 = (b + 1) % DEPTH

        @pl.when(jnp.logical_and(c2 >= 0, c2 < NCH))
        def _():
            pltpu.make_async_copy(h_hbm.at[esds[b2].at[0]], rowss[b2],
                                  gsems[b2]).wait()

            def group(gj, _):
                sl = pl.ds(gj * LANES, LANES)
                dstbs[b2][sl] = esds[b2][1, sl]
                w16 = wbufs[b2][sl]
                for i16 in range(LANES):
                    wj = w16[i16]
                    j = gj * LANES + i16
                    for k in range(FV):
                        fs = pl.ds(k * LANES, LANES)
                        rowss[b2][j, fs] = rowss[b2][j, fs] * wj
                return 0
            lax.fori_loop(0, EB // LANES, group, 0)
            pltpu.async_copy(rowss[b2], acc_sh.at[dstbs[b2]], ssems[b2], add=True)

        @pl.when(c + 1 < NCH)
        def _():
            idx_start(c + 1, b2)

    n_outer = (NCH + 2 + DEPTH - 1) // DEPTH

    def outer(i, _):
        for b in range(DEPTH):
            slot(i, b)
        return 0
    lax.fori_loop(0, n_outer, outer, 0)

    for m in range(NCH - DEPTH, NCH):
        bm = m % DEPTH
        pltpu.make_async_copy(rowss[bm], acc_sh.at[dstbs[bm]], ssems[bm]).wait()
    plsc.subcore_barrier()

    pltpu.sync_copy(acc_sh.at[pl.ds(sid * STRIPE, STRIPE)],
                    out_hbm.at[cid, pl.ds(sid * STRIPE, STRIPE)])
    @pl.when(sid == NS - 1)
    def _():
        pltpu.sync_copy(acc_sh.at[pl.ds(NS * STRIPE, TAIL)],
                        out_hbm.at[cid, pl.ds(NS * STRIPE, TAIL)])


def _sc_aggregate(esd, w, h):
    mesh = plsc.VectorSubcoreMesh(core_axis_name="c", subcore_axis_name="s",
                                  num_cores=NC, num_subcores=NS)
    f = pl.kernel(
        _sc_aggregate_body,
        out_type=jax.ShapeDtypeStruct((NC, N, F), jnp.float32),
        mesh=mesh,
        scratch_types=(
            [pltpu.VMEM_SHARED((N, F), jnp.float32)]
            + [pltpu.VMEM((2, EB), jnp.int32) for _ in range(DEPTH)]
            + [pltpu.VMEM((EB, F), jnp.float32) for _ in range(DEPTH)]
            + [pltpu.VMEM((EB,), jnp.int32) for _ in range(DEPTH)]
            + [pltpu.VMEM((EB,), jnp.float32) for _ in range(DEPTH)]
            + [pltpu.SemaphoreType.DMA for _ in range(3 * DEPTH)]
        ),
    )
    return f(esd, w, h)


GPW = G // NW
SEG_CHUNK = 64
SEG_BUF = SEG_CHUNK + 8


def _sc_segmax_body(p_hbm, batch_hbm, b3_hbm, out_hbm,
                    batch_v, buf0, buf1, b3_v, obuf):
    cid = lax.axis_index("c")
    sid = lax.axis_index("s")
    wid = cid * NS + sid
    g0 = wid * GPW

    pltpu.sync_copy(batch_hbm, batch_v)
    pltpu.sync_copy(b3_hbm, b3_v)

    def cnt(i, carry):
        v = batch_v[pl.ds(i * LANES, LANES)]
        out = []
        for t in range(GPW + 1):
            m = jnp.minimum(jnp.maximum((g0 + t) - v, 0), 1)
            out.append(carry[t] + m)
        return tuple(out)
    zeros = tuple(jnp.zeros((LANES,), jnp.int32) for _ in range(GPW + 1))
    counts = lax.fori_loop(0, N // LANES, cnt, zeros)
    bounds = [sum(c[i] for i in range(LANES)) for c in counts]

    for t in range(GPW):
        s_row = bounds[t]
        e_row = bounds[t + 1]
        acc0 = tuple(jnp.full((LANES,), -jnp.inf, jnp.float32) for _ in range(FV))
        nch = (e_row - s_row + SEG_CHUNK - 1) // SEG_CHUNK

        def chunk(i, acc, s_row=s_row, e_row=e_row):
            base = s_row + i * SEG_CHUNK
            db = jnp.minimum((base // 8) * 8, N - SEG_BUF)
            pltpu.sync_copy(p_hbm.at[0, pl.ds(db, SEG_BUF)], buf0)
            pltpu.sync_copy(p_hbm.at[1, pl.ds(db, SEG_BUF)], buf1)
            off = base - db
            cnt_rows = jnp.minimum(e_row - base, SEG_CHUNK)

            def row(r, acc):
                rr = off + r
                new = []
                for k in range(FV):
                    sl = pl.ds(k * LANES, LANES)
                    new.append(jnp.maximum(acc[k], buf0[rr, sl] + buf1[rr, sl]))
                return tuple(new)
            return lax.fori_loop(0, cnt_rows, row, acc)

        acc = lax.fori_loop(0, nch, chunk, acc0)
        for k in range(FV):
            obuf[pl.ds(t * F + k * LANES, LANES)] = acc[k] + b3_v[pl.ds(k * LANES, LANES)]
    pltpu.sync_copy(obuf, out_hbm.at[pl.ds(wid * GPW * F, GPW * F)])


def _sc_segmax(p, batch, b3):
    mesh = plsc.VectorSubcoreMesh(core_axis_name="c", subcore_axis_name="s",
                                  num_cores=NC, num_subcores=NS)
    f = pl.kernel(
        _sc_segmax_body,
        out_type=jax.ShapeDtypeStruct((G * F,), jnp.float32),
        mesh=mesh,
        scratch_types=[
            pltpu.VMEM((N,), jnp.int32),
            pltpu.VMEM((SEG_BUF, F), jnp.float32),
            pltpu.VMEM((SEG_BUF, F), jnp.float32),
            pltpu.VMEM((F,), jnp.float32),
            pltpu.VMEM((GPW * F,), jnp.float32),
        ],
    )
    return f(p, batch, b3).reshape(G, F)


MM_ROWS = 2000


def _mm_first_body(x_ref, w_ref, o_ref):
    o_ref[...] = jnp.dot(x_ref[...], w_ref[...], preferred_element_type=jnp.float32)


def _mm_first(x, w):
    grid = (N // MM_ROWS,)
    return pl.pallas_call(
        _mm_first_body,
        grid=grid,
        in_specs=[
            pl.BlockSpec((MM_ROWS, F), lambda i: (i, 0)),
            pl.BlockSpec((F, F), lambda i: (0, 0)),
        ],
        out_specs=pl.BlockSpec((MM_ROWS, F), lambda i: (i, 0)),
        out_shape=jax.ShapeDtypeStruct((N, F), jnp.float32),
    )(x, w)


def _mm_fused_body(p_ref, b_ref, w_ref, o_ref):
    h = jax.nn.relu(p_ref[0] + p_ref[1] + b_ref[...])
    o_ref[...] = jnp.dot(h, w_ref[...], preferred_element_type=jnp.float32)


def _mm_fused(p, b, w):
    grid = (N // MM_ROWS,)
    return pl.pallas_call(
        _mm_fused_body,
        grid=grid,
        in_specs=[
            pl.BlockSpec((NC, MM_ROWS, F), lambda i: (0, i, 0)),
            pl.BlockSpec((1, F), lambda i: (0, 0)),
            pl.BlockSpec((F, F), lambda i: (0, 0)),
        ],
        out_specs=pl.BlockSpec((MM_ROWS, F), lambda i: (i, 0)),
        out_shape=jax.ShapeDtypeStruct((N, F), jnp.float32),
    )(p, b.reshape(1, F), w)


def _head_body(g_ref, gam_ref, bet_ref, lw_ref, lb_ref, o_ref):
    g = g_ref[...]
    mean = jnp.mean(g, axis=0, keepdims=True)
    var = jnp.mean((g - mean) ** 2, axis=0, keepdims=True)
    gn = (g - mean) * lax.rsqrt(var + 1e-5) * gam_ref[...] + bet_ref[...]
    o_ref[...] = jnp.sum(gn * lw_ref[...], axis=1, keepdims=True) + lb_ref[...]


def _head(g, gamma, beta, lin_W, lin_b):
    return pl.pallas_call(
        _head_body,
        out_shape=jax.ShapeDtypeStruct((G, 1), jnp.float32),
    )(g, gamma.reshape(1, F), beta.reshape(1, F), lin_W.reshape(1, F),
      lin_b.reshape(1, 1))


def kernel(x, edge_index, edge_weight, batch, W1, b1, W2, b2, W3, b3,
           bn_gamma, bn_beta, lin_W, lin_b):
    esd = jnp.pad(edge_index, ((0, 0), (0, EPAD - E)))
    wpad = jnp.pad(edge_weight, (0, EPAD - E))
    h = _mm_first(x, W1)
    p = _sc_aggregate(esd, wpad, h)
    h = _mm_fused(p, b1, W2)
    p = _sc_aggregate(esd, wpad, h)
    h = _mm_fused(p, b2, W3)
    p = _sc_aggregate(esd, wpad, h)
    g = _sc_segmax(p, batch, b3)
    return _head(g, bn_gamma, bn_beta, lin_W, lin_b)


# speedup vs baseline: 5.9452x; 1.0019x over previous
import functools

import jax
import jax.numpy as jnp
from jax import lax
from jax.experimental import pallas as pl
from jax.experimental.pallas import tpu as pltpu
from jax.experimental.pallas import tpu_sc as plsc

N = 10000
E = 320000
F = 128
G = 64

NC = 2
NS = 16
NW = NC * NS
LANES = 16

EB = 128
NUM_CHUNKS = E // EB
STRIPE = 624
TAIL = N - NS * STRIPE
ZROWS = 208
FV = F // LANES


NCH = (NUM_CHUNKS + NW - 1) // NW
EPAD = NW * NCH * EB
DEPTH = 3


def _sc_aggregate_body(esd_hbm, w_hbm, h_hbm, out_hbm, acc_sh, *rest):
    esds = rest[0:DEPTH]
    rowss = rest[DEPTH:2 * DEPTH]
    dstbs = rest[2 * DEPTH:3 * DEPTH]
    wbufs = rest[3 * DEPTH:4 * DEPTH]
    isems = rest[4 * DEPTH:5 * DEPTH]
    gsems = rest[5 * DEPTH:6 * DEPTH]
    ssems = rest[6 * DEPTH:7 * DEPTH]

    cid = lax.axis_index("c")
    sid = lax.axis_index("s")
    wid = cid * NS + sid

    def idx_src(c):
        return esd_hbm.at[:, pl.ds((wid + c * NW) * EB, EB)]

    def w_src(c):
        return w_hbm.at[pl.ds((wid + c * NW) * EB, EB)]

    def idx_start(c, b):
        pltpu.async_copy(idx_src(c), esds[b], isems[b])
        pltpu.async_copy(w_src(c), wbufs[b], isems[b])

    def idx_wait(c, b):
        pltpu.make_async_copy(idx_src(c), esds[b], isems[b]).wait()
        pltpu.make_async_copy(w_src(c), wbufs[b], isems[b]).wait()

    idx_start(0, 0)

    def zrow(r, _):
        for k in range(FV):
            rowss[0][r, pl.ds(k * LANES, LANES)] = jnp.zeros((LANES,), jnp.float32)
        return 0
    lax.fori_loop(0, EB, zrow, 0)
    for j in range(STRIPE // EB):
        pltpu.sync_copy(rowss[0], acc_sh.at[pl.ds(sid * STRIPE + j * EB, EB)])
    rem = STRIPE - (STRIPE // EB) * EB
    pltpu.sync_copy(rowss[0].at[pl.ds(0, rem)],
                    acc_sh.at[pl.ds(sid * STRIPE + (STRIPE // EB) * EB, rem)])
    @pl.when(sid == NS - 1)
    def _():
        pltpu.sync_copy(rowss[0].at[pl.ds(0, TAIL)],
                        acc_sh.at[pl.ds(NS * STRIPE, TAIL)])
    plsc.subcore_barrier()

    def slot(i, b):
        c = i * DEPTH + b

        @pl.when(jnp.logical_and(c < NCH, c >= DEPTH))
        def _():
            pltpu.make_async_copy(rowss[b], acc_sh.at[dstbs[b]], ssems[b]).wait()

        @pl.when(c < NCH)
        def _():
            idx_wait(c, b)
            pltpu.async_copy(h_hbm.at[esds[b].at[0]], rowss[b], gsems[b])

        c2 = c - 2
        b2 = (b + 1) % DEPTH

        @pl.when(jnp.logical_and(c2 >= 0, c2 < NCH))
        def _():
            pltpu.make_async_copy(h_hbm.at[esds[b2].at[0]], rowss[b2],
                                  gsems[b2]).wait()

            def group(gj, _):
                sl = pl.ds(gj * LANES, LANES)
                dstbs[b2][sl] = esds[b2][1, sl]
                w16 = wbufs[b2][sl]
                for i16 in range(LANES):
                    wj = w16[i16]
                    j = gj * LANES + i16
                    for k in range(FV):
                        fs = pl.ds(k * LANES, LANES)
                        rowss[b2][j, fs] = rowss[b2][j, fs] * wj
                return 0
            lax.fori_loop(0, EB // LANES, group, 0)
            pltpu.async_copy(rowss[b2], acc_sh.at[dstbs[b2]], ssems[b2], add=True)

        @pl.when(c + 1 < NCH)
        def _():
            idx_start(c + 1, b2)

    n_outer = (NCH + 2 + DEPTH - 1) // DEPTH

    def outer(i, _):
        for b in range(DEPTH):
            slot(i, b)
        return 0
    lax.fori_loop(0, n_outer, outer, 0)

    for m in range(NCH - DEPTH, NCH):
        bm = m % DEPTH
        pltpu.make_async_copy(rowss[bm], acc_sh.at[dstbs[bm]], ssems[bm]).wait()
    plsc.subcore_barrier()

    pltpu.sync_copy(acc_sh.at[pl.ds(sid * STRIPE, STRIPE)],
                    out_hbm.at[cid, pl.ds(sid * STRIPE, STRIPE)])
    @pl.when(sid == NS - 1)
    def _():
        pltpu.sync_copy(acc_sh.at[pl.ds(NS * STRIPE, TAIL)],
                        out_hbm.at[cid, pl.ds(NS * STRIPE, TAIL)])


def _sc_aggregate(esd, w, h):
    mesh = plsc.VectorSubcoreMesh(core_axis_name="c", subcore_axis_name="s",
                                  num_cores=NC, num_subcores=NS)
    f = pl.kernel(
        _sc_aggregate_body,
        out_type=jax.ShapeDtypeStruct((NC, N, F), jnp.float32),
        mesh=mesh,
        compiler_params=pltpu.CompilerParams(use_tc_tiling_on_sc=False,
                                             needs_layout_passes=False),
        scratch_types=(
            [pltpu.VMEM_SHARED((N, F), jnp.float32)]
            + [pltpu.VMEM((2, EB), jnp.int32) for _ in range(DEPTH)]
            + [pltpu.VMEM((EB, F), jnp.float32) for _ in range(DEPTH)]
            + [pltpu.VMEM((EB,), jnp.int32) for _ in range(DEPTH)]
            + [pltpu.VMEM((EB,), jnp.float32) for _ in range(DEPTH)]
            + [pltpu.SemaphoreType.DMA for _ in range(3 * DEPTH)]
        ),
    )
    return f(esd, w, h)


GPW = G // NW
SEG_CHUNK = 64
SEG_BUF = SEG_CHUNK + 8


def _sc_segmax_body(p_hbm, batch_hbm, b3_hbm, out_hbm,
                    batch_v, buf0, buf1, b3_v, obuf):
    cid = lax.axis_index("c")
    sid = lax.axis_index("s")
    wid = cid * NS + sid
    g0 = wid * GPW

    pltpu.sync_copy(batch_hbm, batch_v)
    pltpu.sync_copy(b3_hbm, b3_v)

    def cnt(i, carry):
        v = batch_v[pl.ds(i * LANES, LANES)]
        out = []
        for t in range(GPW + 1):
            m = jnp.minimum(jnp.maximum((g0 + t) - v, 0), 1)
            out.append(carry[t] + m)
        return tuple(out)
    zeros = tuple(jnp.zeros((LANES,), jnp.int32) for _ in range(GPW + 1))
    counts = lax.fori_loop(0, N // LANES, cnt, zeros)
    bounds = [sum(c[i] for i in range(LANES)) for c in counts]

    for t in range(GPW):
        s_row = bounds[t]
        e_row = bounds[t + 1]
        acc0 = tuple(jnp.full((LANES,), -jnp.inf, jnp.float32) for _ in range(FV))
        nch = (e_row - s_row + SEG_CHUNK - 1) // SEG_CHUNK

        def chunk(i, acc, s_row=s_row, e_row=e_row):
            base = s_row + i * SEG_CHUNK
            db = jnp.minimum((base // 8) * 8, N - SEG_BUF)
            pltpu.sync_copy(p_hbm.at[0, pl.ds(db, SEG_BUF)], buf0)
            pltpu.sync_copy(p_hbm.at[1, pl.ds(db, SEG_BUF)], buf1)
            off = base - db
            cnt_rows = jnp.minimum(e_row - base, SEG_CHUNK)

            def row(r, acc):
                rr = off + r
                new = []
                for k in range(FV):
                    sl = pl.ds(k * LANES, LANES)
                    new.append(jnp.maximum(acc[k], buf0[rr, sl] + buf1[rr, sl]))
                return tuple(new)
            return lax.fori_loop(0, cnt_rows, row, acc)

        acc = lax.fori_loop(0, nch, chunk, acc0)
        for k in range(FV):
            obuf[pl.ds(t * F + k * LANES, LANES)] = acc[k] + b3_v[pl.ds(k * LANES, LANES)]
    pltpu.sync_copy(obuf, out_hbm.at[pl.ds(wid * GPW * F, GPW * F)])


def _sc_segmax(p, batch, b3):
    mesh = plsc.VectorSubcoreMesh(core_axis_name="c", subcore_axis_name="s",
                                  num_cores=NC, num_subcores=NS)
    f = pl.kernel(
        _sc_segmax_body,
        out_type=jax.ShapeDtypeStruct((G * F,), jnp.float32),
        mesh=mesh,
        scratch_types=[
            pltpu.VMEM((N,), jnp.int32),
            pltpu.VMEM((SEG_BUF, F), jnp.float32),
            pltpu.VMEM((SEG_BUF, F), jnp.float32),
            pltpu.VMEM((F,), jnp.float32),
            pltpu.VMEM((GPW * F,), jnp.float32),
        ],
    )
    return f(p, batch, b3).reshape(G, F)


MM_ROWS = 2000


def _mm_first_body(x_ref, w_ref, o_ref):
    o_ref[...] = jnp.dot(x_ref[...], w_ref[...], preferred_element_type=jnp.float32)


def _mm_first(x, w):
    grid = (N // MM_ROWS,)
    return pl.pallas_call(
        _mm_first_body,
        grid=grid,
        in_specs=[
            pl.BlockSpec((MM_ROWS, F), lambda i: (i, 0)),
            pl.BlockSpec((F, F), lambda i: (0, 0)),
        ],
        out_specs=pl.BlockSpec((MM_ROWS, F), lambda i: (i, 0)),
        out_shape=jax.ShapeDtypeStruct((N, F), jnp.float32),
    )(x, w)


def _mm_fused_body(p_ref, b_ref, w_ref, o_ref):
    h = jax.nn.relu(p_ref[0] + p_ref[1] + b_ref[...])
    o_ref[...] = jnp.dot(h, w_ref[...], preferred_element_type=jnp.float32)


def _mm_fused(p, b, w):
    grid = (N // MM_ROWS,)
    return pl.pallas_call(
        _mm_fused_body,
        grid=grid,
        in_specs=[
            pl.BlockSpec((NC, MM_ROWS, F), lambda i: (0, i, 0)),
            pl.BlockSpec((1, F), lambda i: (0, 0)),
            pl.BlockSpec((F, F), lambda i: (0, 0)),
        ],
        out_specs=pl.BlockSpec((MM_ROWS, F), lambda i: (i, 0)),
        out_shape=jax.ShapeDtypeStruct((N, F), jnp.float32),
    )(p, b.reshape(1, F), w)


def _head_body(g_ref, gam_ref, bet_ref, lw_ref, lb_ref, o_ref):
    g = g_ref[...]
    mean = jnp.mean(g, axis=0, keepdims=True)
    var = jnp.mean((g - mean) ** 2, axis=0, keepdims=True)
    gn = (g - mean) * lax.rsqrt(var + 1e-5) * gam_ref[...] + bet_ref[...]
    o_ref[...] = jnp.sum(gn * lw_ref[...], axis=1, keepdims=True) + lb_ref[...]


def _head(g, gamma, beta, lin_W, lin_b):
    return pl.pallas_call(
        _head_body,
        out_shape=jax.ShapeDtypeStruct((G, 1), jnp.float32),
    )(g, gamma.reshape(1, F), beta.reshape(1, F), lin_W.reshape(1, F),
      lin_b.reshape(1, 1))


def kernel(x, edge_index, edge_weight, batch, W1, b1, W2, b2, W3, b3,
           bn_gamma, bn_beta, lin_W, lin_b):
    esd = jnp.pad(edge_index, ((0, 0), (0, EPAD - E)))
    wpad = jnp.pad(edge_weight, (0, EPAD - E))
    h = _mm_first(x, W1)
    p = _sc_aggregate(esd, wpad, h)
    h = _mm_fused(p, b1, W2)
    p = _sc_aggregate(esd, wpad, h)
    h = _mm_fused(p, b2, W3)
    p = _sc_aggregate(esd, wpad, h)
    g = _sc_segmax(p, batch, b3)
    return _head(g, bn_gamma, bn_beta, lin_W, lin_b)


# speedup vs baseline: 7.5884x; 1.2764x over previous
import functools

import jax
import jax.numpy as jnp
import numpy as np
from jax import lax
from jax.experimental import pallas as pl
from jax.experimental.pallas import tpu as pltpu
from jax.experimental.pallas import tpu_sc as plsc

N = 10000
E = 320000
F = 128
G = 64

NC = 2
NS = 16
NW = NC * NS
LANES = 16

EB = 128
NUM_CHUNKS = E // EB
STRIPE = 624
TAIL = N - NS * STRIPE
ZROWS = 208
FV = F // LANES


NCH = (NUM_CHUNKS + NW - 1) // NW
EPAD = NW * NCH * EB
DEPTH = 3


def _sc_aggregate_body(esd_hbm, w_hbm, h_hbm, out_hbm, acc_sh, *rest):
    esds = rest[0:DEPTH]
    rowss = rest[DEPTH:2 * DEPTH]
    dstbs = rest[2 * DEPTH:3 * DEPTH]
    wbufs = rest[3 * DEPTH:4 * DEPTH]
    isems = rest[4 * DEPTH:5 * DEPTH]
    gsems = rest[5 * DEPTH:6 * DEPTH]
    ssems = rest[6 * DEPTH:7 * DEPTH]

    cid = lax.axis_index("c")
    sid = lax.axis_index("s")
    wid = cid * NS + sid

    def idx_src(c):
        return esd_hbm.at[:, pl.ds((wid + c * NW) * EB, EB)]

    def w_src(c):
        return w_hbm.at[pl.ds((wid + c * NW) * EB, EB)]

    def idx_start(c, b):
        pltpu.async_copy(idx_src(c), esds[b], isems[b])
        pltpu.async_copy(w_src(c), wbufs[b], isems[b])

    def idx_wait(c, b):
        pltpu.make_async_copy(idx_src(c), esds[b], isems[b]).wait()
        pltpu.make_async_copy(w_src(c), wbufs[b], isems[b]).wait()

    idx_start(0, 0)

    def zrow(r, _):
        for k in range(FV):
            rowss[0][r, pl.ds(k * LANES, LANES)] = jnp.zeros((LANES,), jnp.float32)
        return 0
    lax.fori_loop(0, EB, zrow, 0)
    for j in range(STRIPE // EB):
        pltpu.sync_copy(rowss[0], acc_sh.at[pl.ds(sid * STRIPE + j * EB, EB)])
    rem = STRIPE - (STRIPE // EB) * EB
    pltpu.sync_copy(rowss[0].at[pl.ds(0, rem)],
                    acc_sh.at[pl.ds(sid * STRIPE + (STRIPE // EB) * EB, rem)])
    @pl.when(sid == NS - 1)
    def _():
        pltpu.sync_copy(rowss[0].at[pl.ds(0, TAIL)],
                        acc_sh.at[pl.ds(NS * STRIPE, TAIL)])
    plsc.subcore_barrier()

    def slot(i, b):
        c = i * DEPTH + b

        @pl.when(jnp.logical_and(c < NCH, c >= DEPTH))
        def _():
            pltpu.make_async_copy(rowss[b], acc_sh.at[dstbs[b]], ssems[b]).wait()

        @pl.when(c < NCH)
        def _():
            idx_wait(c, b)
            pltpu.async_copy(h_hbm.at[esds[b].at[0]], rowss[b], gsems[b])

        c2 = c - 2
        b2 = (b + 1) % DEPTH

        @pl.when(jnp.logical_and(c2 >= 0, c2 < NCH))
        def _():
            pltpu.make_async_copy(h_hbm.at[esds[b2].at[0]], rowss[b2],
                                  gsems[b2]).wait()

            def group(gj, _):
                sl = pl.ds(gj * LANES, LANES)
                dstbs[b2][sl] = esds[b2][1, sl]
                w16 = wbufs[b2][sl]
                for i16 in range(LANES):
                    wj = w16[i16]
                    j = gj * LANES + i16
                    for k in range(FV):
                        fs = pl.ds(k * LANES, LANES)
                        rowss[b2][j, fs] = rowss[b2][j, fs] * wj
                return 0
            lax.fori_loop(0, EB // LANES, group, 0)
            pltpu.async_copy(rowss[b2], acc_sh.at[dstbs[b2]], ssems[b2], add=True)

        @pl.when(c + 1 < NCH)
        def _():
            idx_start(c + 1, b2)

    n_outer = (NCH + 2 + DEPTH - 1) // DEPTH

    def outer(i, _):
        for b in range(DEPTH):
            slot(i, b)
        return 0
    lax.fori_loop(0, n_outer, outer, 0)

    for m in range(NCH - DEPTH, NCH):
        bm = m % DEPTH
        pltpu.make_async_copy(rowss[bm], acc_sh.at[dstbs[bm]], ssems[bm]).wait()
    plsc.subcore_barrier()

    pltpu.sync_copy(acc_sh.at[pl.ds(sid * STRIPE, STRIPE)],
                    out_hbm.at[cid, pl.ds(sid * STRIPE, STRIPE)])
    @pl.when(sid == NS - 1)
    def _():
        pltpu.sync_copy(acc_sh.at[pl.ds(NS * STRIPE, TAIL)],
                        out_hbm.at[cid, pl.ds(NS * STRIPE, TAIL)])


def _sc_aggregate(esd, w, h):
    mesh = plsc.VectorSubcoreMesh(core_axis_name="c", subcore_axis_name="s",
                                  num_cores=NC, num_subcores=NS)
    f = pl.kernel(
        _sc_aggregate_body,
        out_type=jax.ShapeDtypeStruct((NC, N, F), jnp.float32),
        mesh=mesh,
        compiler_params=pltpu.CompilerParams(use_tc_tiling_on_sc=False,
                                             needs_layout_passes=False),
        scratch_types=(
            [pltpu.VMEM_SHARED((N, F), jnp.float32)]
            + [pltpu.VMEM((2, EB), jnp.int32) for _ in range(DEPTH)]
            + [pltpu.VMEM((EB, F), jnp.float32) for _ in range(DEPTH)]
            + [pltpu.VMEM((EB,), jnp.int32) for _ in range(DEPTH)]
            + [pltpu.VMEM((EB,), jnp.float32) for _ in range(DEPTH)]
            + [pltpu.SemaphoreType.DMA for _ in range(3 * DEPTH)]
        ),
    )
    return f(esd, w, h)


EB_B = 96
NUM_CHUNKS_B = -(-E // EB_B)
FH = F // 2
FHV = FH // LANES
NCH_B = (NUM_CHUNKS_B + NW - 1) // NW
EPAD_B = NW * NCH_B * EB_B
DEPTH_B = 4
SB = 2

_PERM = np.concatenate(
    [np.concatenate([np.arange(0, 32, 2), np.arange(1, 32, 2)]) + 32 * k
     for k in range(F // 32)])


def _sc_aggregate_bf_body(esd_hbm, w_hbm, h_hbm, out_hbm, acc_sh, *rest):
    esds = rest[0:DEPTH_B]
    rowss = rest[DEPTH_B:2 * DEPTH_B]
    wbufs = rest[2 * DEPTH_B:3 * DEPTH_B]
    sbufs = rest[3 * DEPTH_B:3 * DEPTH_B + SB]
    dstbs = rest[3 * DEPTH_B + SB:3 * DEPTH_B + 2 * SB]
    isems = rest[3 * DEPTH_B + 2 * SB:4 * DEPTH_B + 2 * SB]
    gsems = rest[4 * DEPTH_B + 2 * SB:5 * DEPTH_B + 2 * SB]
    ssems = rest[5 * DEPTH_B + 2 * SB:5 * DEPTH_B + 3 * SB]

    cid = lax.axis_index("c")
    sid = lax.axis_index("s")
    wid = cid * NS + sid

    def idx_src(c):
        return esd_hbm.at[:, pl.ds((wid + c * NW) * EB_B, EB_B)]

    def w_src(c):
        return w_hbm.at[pl.ds((wid + c * NW) * EB_B, EB_B)]

    def idx_start(c, b):
        pltpu.async_copy(idx_src(c), esds[b], isems[b])
        pltpu.async_copy(w_src(c), wbufs[b], isems[b])

    def idx_wait(c, b):
        pltpu.make_async_copy(idx_src(c), esds[b], isems[b]).wait()
        pltpu.make_async_copy(w_src(c), wbufs[b], isems[b]).wait()

    idx_start(0, 0)

    def zrow(r, _):
        for k in range(FV):
            sbufs[0][r, pl.ds(k * LANES, LANES)] = jnp.zeros((LANES,), jnp.float32)
        return 0
    lax.fori_loop(0, EB_B, zrow, 0)
    for j in range(STRIPE // EB_B):
        pltpu.sync_copy(sbufs[0], acc_sh.at[pl.ds(sid * STRIPE + j * EB_B, EB_B)])
    rem = STRIPE - (STRIPE // EB_B) * EB_B
    pltpu.sync_copy(sbufs[0].at[pl.ds(0, rem)],
                    acc_sh.at[pl.ds(sid * STRIPE + (STRIPE // EB_B) * EB_B, rem)])
    @pl.when(sid == NS - 1)
    def _():
        pltpu.sync_copy(sbufs[0].at[pl.ds(0, TAIL)],
                        acc_sh.at[pl.ds(NS * STRIPE, TAIL)])
    plsc.subcore_barrier()

    def slot(i, b):
        c = i * DEPTH_B + b

        @pl.when(c < NCH_B)
        def _():
            idx_wait(c, b)
            pltpu.async_copy(h_hbm.at[esds[b].at[0]], rowss[b], gsems[b])

        c2 = c - (DEPTH_B - 1)
        b2 = (b + 1) % DEPTH_B
        q = (b + 1) % SB

        @pl.when(jnp.logical_and(c2 >= 2, c2 < NCH_B))
        def _():
            pltpu.make_async_copy(sbufs[q], acc_sh.at[dstbs[q]], ssems[q]).wait()

        @pl.when(jnp.logical_and(c2 >= 0, c2 < NCH_B))
        def _():
            pltpu.make_async_copy(h_hbm.at[esds[b2].at[0]], rowss[b2],
                                  gsems[b2]).wait()

            def group(gj, _):
                sl = pl.ds(gj * LANES, LANES)
                dstbs[q][sl] = esds[b2][1, sl]
                w16 = wbufs[b2][sl]
                for i16 in range(LANES):
                    wj = w16[i16]
                    j = gj * LANES + i16
                    for k in range(FHV):
                        packed = rowss[b2][j, pl.ds(k * LANES, LANES)]
                        pb = plsc.bitcast(packed, jnp.bfloat16)
                        ev, od = plsc.unpack(pb, format=plsc.PackFormat.INTERLEAVED)
                        sbufs[q][j, pl.ds(k * 2 * LANES, LANES)] = ev * wj
                        sbufs[q][j, pl.ds((k * 2 + 1) * LANES, LANES)] = od * wj
                return 0
            lax.fori_loop(0, EB_B // LANES, group, 0)
            pltpu.async_copy(sbufs[q], acc_sh.at[dstbs[q]], ssems[q], add=True)

        @pl.when(c + 1 < NCH_B)
        def _():
            idx_start(c + 1, b2)

    n_outer = (NCH_B + 2 * (DEPTH_B - 1) + DEPTH_B - 1) // DEPTH_B

    def outer(i, _):
        for b in range(DEPTH_B):
            slot(i, b)
        return 0
    lax.fori_loop(0, n_outer, outer, 0)

    for m in range(NCH_B - SB, NCH_B):
        qm = m % SB
        pltpu.make_async_copy(sbufs[qm], acc_sh.at[dstbs[qm]], ssems[qm]).wait()
    plsc.subcore_barrier()

    pltpu.sync_copy(acc_sh.at[pl.ds(sid * STRIPE, STRIPE)],
                    out_hbm.at[cid, pl.ds(sid * STRIPE, STRIPE)])
    @pl.when(sid == NS - 1)
    def _():
        pltpu.sync_copy(acc_sh.at[pl.ds(NS * STRIPE, TAIL)],
                        out_hbm.at[cid, pl.ds(NS * STRIPE, TAIL)])


def _sc_aggregate_bf(esd, w, h):
    mesh = plsc.VectorSubcoreMesh(core_axis_name="c", subcore_axis_name="s",
                                  num_cores=NC, num_subcores=NS)
    f = pl.kernel(
        _sc_aggregate_bf_body,
        out_type=jax.ShapeDtypeStruct((NC, N, F), jnp.float32),
        mesh=mesh,
        compiler_params=pltpu.CompilerParams(use_tc_tiling_on_sc=False,
                                             needs_layout_passes=False),
        scratch_types=(
            [pltpu.VMEM_SHARED((N, F), jnp.float32)]
            + [pltpu.VMEM((2, EB_B), jnp.int32) for _ in range(DEPTH_B)]
            + [pltpu.VMEM((EB_B, FH), jnp.int32) for _ in range(DEPTH_B)]
            + [pltpu.VMEM((EB_B,), jnp.float32) for _ in range(DEPTH_B)]
            + [pltpu.VMEM((EB_B, F), jnp.float32) for _ in range(SB)]
            + [pltpu.VMEM((EB_B,), jnp.int32) for _ in range(SB)]
            + [pltpu.SemaphoreType.DMA for _ in range(2 * DEPTH_B + SB)]
        ),
    )
    return f(esd, w, h)


GPW = G // NW
SEG_CHUNK = 64
SEG_BUF = SEG_CHUNK + 8


def _sc_segmax_body(p_hbm, batch_hbm, b3_hbm, out_hbm,
                    batch_v, buf0, buf1, b3_v, obuf):
    cid = lax.axis_index("c")
    sid = lax.axis_index("s")
    wid = cid * NS + sid
    g0 = wid * GPW

    pltpu.sync_copy(batch_hbm, batch_v)
    pltpu.sync_copy(b3_hbm, b3_v)

    def cnt(i, carry):
        v = batch_v[pl.ds(i * LANES, LANES)]
        out = []
        for t in range(GPW + 1):
            m = jnp.minimum(jnp.maximum((g0 + t) - v, 0), 1)
            out.append(carry[t] + m)
        return tuple(out)
    zeros = tuple(jnp.zeros((LANES,), jnp.int32) for _ in range(GPW + 1))
    counts = lax.fori_loop(0, N // LANES, cnt, zeros)
    bounds = [sum(c[i] for i in range(LANES)) for c in counts]

    for t in range(GPW):
        s_row = bounds[t]
        e_row = bounds[t + 1]
        acc0 = tuple(jnp.full((LANES,), -jnp.inf, jnp.float32) for _ in range(FV))
        nch = (e_row - s_row + SEG_CHUNK - 1) // SEG_CHUNK

        def chunk(i, acc, s_row=s_row, e_row=e_row):
            base = s_row + i * SEG_CHUNK
            db = jnp.minimum((base // 8) * 8, N - SEG_BUF)
            pltpu.sync_copy(p_hbm.at[0, pl.ds(db, SEG_BUF)], buf0)
            pltpu.sync_copy(p_hbm.at[1, pl.ds(db, SEG_BUF)], buf1)
            off = base - db
            cnt_rows = jnp.minimum(e_row - base, SEG_CHUNK)

            def row(r, acc):
                rr = off + r
                new = []
                for k in range(FV):
                    sl = pl.ds(k * LANES, LANES)
                    new.append(jnp.maximum(acc[k], buf0[rr, sl] + buf1[rr, sl]))
                return tuple(new)
            return lax.fori_loop(0, cnt_rows, row, acc)

        acc = lax.fori_loop(0, nch, chunk, acc0)
        for k in range(FV):
            obuf[pl.ds(t * F + k * LANES, LANES)] = acc[k] + b3_v[pl.ds(k * LANES, LANES)]
    pltpu.sync_copy(obuf, out_hbm.at[pl.ds(wid * GPW * F, GPW * F)])


def _sc_segmax(p, batch, b3):
    mesh = plsc.VectorSubcoreMesh(core_axis_name="c", subcore_axis_name="s",
                                  num_cores=NC, num_subcores=NS)
    f = pl.kernel(
        _sc_segmax_body,
        out_type=jax.ShapeDtypeStruct((G * F,), jnp.float32),
        mesh=mesh,
        scratch_types=[
            pltpu.VMEM((N,), jnp.int32),
            pltpu.VMEM((SEG_BUF, F), jnp.float32),
            pltpu.VMEM((SEG_BUF, F), jnp.float32),
            pltpu.VMEM((F,), jnp.float32),
            pltpu.VMEM((GPW * F,), jnp.float32),
        ],
    )
    return f(p, batch, b3).reshape(G, F)


MM_ROWS = 2000


def _mm_first_body(x_ref, w_ref, o_ref):
    o_ref[...] = jnp.dot(x_ref[...], w_ref[...], preferred_element_type=jnp.float32)


def _mm_first(x, w):
    grid = (N // MM_ROWS,)
    return pl.pallas_call(
        _mm_first_body,
        grid=grid,
        in_specs=[
            pl.BlockSpec((MM_ROWS, F), lambda i: (i, 0)),
            pl.BlockSpec((F, F), lambda i: (0, 0)),
        ],
        out_specs=pl.BlockSpec((MM_ROWS, F), lambda i: (i, 0)),
        out_shape=jax.ShapeDtypeStruct((N, F), jnp.float32),
    )(x, w)


def _mm_fused_body(p_ref, b_ref, w_ref, o_ref):
    h = jax.nn.relu(p_ref[0] + p_ref[1] + b_ref[...])
    o_ref[...] = jnp.dot(h, w_ref[...], preferred_element_type=jnp.float32)


def _mm_fused(p, b, w):
    grid = (N // MM_ROWS,)
    return pl.pallas_call(
        _mm_fused_body,
        grid=grid,
        in_specs=[
            pl.BlockSpec((NC, MM_ROWS, F), lambda i: (0, i, 0)),
            pl.BlockSpec((1, F), lambda i: (0, 0)),
            pl.BlockSpec((F, F), lambda i: (0, 0)),
        ],
        out_specs=pl.BlockSpec((MM_ROWS, F), lambda i: (i, 0)),
        out_shape=jax.ShapeDtypeStruct((N, F), jnp.float32),
    )(p, b.reshape(1, F), w)


def _mm_fused_bf_body(p_ref, b_ref, w_ref, o_ref):
    h = jax.nn.relu(p_ref[0] + p_ref[1] + b_ref[...])
    o_ref[...] = jnp.dot(h, w_ref[...],
                         preferred_element_type=jnp.float32).astype(jnp.bfloat16)


def _mm_fused_bf(p, b, w):
    grid = (N // MM_ROWS,)
    return pl.pallas_call(
        _mm_fused_bf_body,
        grid=grid,
        in_specs=[
            pl.BlockSpec((NC, MM_ROWS, F), lambda i: (0, i, 0)),
            pl.BlockSpec((1, F), lambda i: (0, 0)),
            pl.BlockSpec((F, F), lambda i: (0, 0)),
        ],
        out_specs=pl.BlockSpec((MM_ROWS, F), lambda i: (i, 0)),
        out_shape=jax.ShapeDtypeStruct((N, F), jnp.bfloat16),
    )(p, b.reshape(1, F), w)


def _head_body(g_ref, gam_ref, bet_ref, lw_ref, lb_ref, o_ref):
    g = g_ref[...]
    mean = jnp.mean(g, axis=0, keepdims=True)
    var = jnp.mean((g - mean) ** 2, axis=0, keepdims=True)
    gn = (g - mean) * lax.rsqrt(var + 1e-5) * gam_ref[...] + bet_ref[...]
    o_ref[...] = jnp.sum(gn * lw_ref[...], axis=1, keepdims=True) + lb_ref[...]


def _head(g, gamma, beta, lin_W, lin_b):
    return pl.pallas_call(
        _head_body,
        out_shape=jax.ShapeDtypeStruct((G, 1), jnp.float32),
    )(g, gamma.reshape(1, F), beta.reshape(1, F), lin_W.reshape(1, F),
      lin_b.reshape(1, 1))


def kernel(x, edge_index, edge_weight, batch, W1, b1, W2, b2, W3, b3,
           bn_gamma, bn_beta, lin_W, lin_b):
    def pad_edges(epad):
        padn = epad - E
        pidx = (jnp.arange(padn, dtype=jnp.int32) * 37) % N
        esd = jnp.concatenate([edge_index, jnp.stack([pidx, pidx])], axis=1)
        return esd, jnp.pad(edge_weight, (0, padn))
    esd, wpad = pad_edges(EPAD)
    esd_b, wpad_b = pad_edges(EPAD_B)

    perm = jnp.asarray(_PERM)
    W3p, b2p = W3[perm, :], b2[perm]

    def as_i32(hb):
        return lax.bitcast_convert_type(hb.reshape(N, FH, 2), jnp.int32)

    h = _mm_first(x, W1)
    p = _sc_aggregate(esd, wpad, h)
    h = as_i32(_mm_fused_bf(p, b1, W2))
    p = _sc_aggregate_bf(esd_b, wpad_b, h)
    h = _mm_fused(p, b2p, W3p)
    p = _sc_aggregate(esd, wpad, h)
    g = _sc_segmax(p, batch, b3)
    return _head(g, bn_gamma, bn_beta, lin_W, lin_b)
